# baseline (speedup 1.0000x reference)
import functools

import jax
import jax.numpy as jnp
from jax import lax
from jax.experimental import pallas as pl
from jax.experimental.pallas import tpu as pltpu
from jax.experimental.pallas import tpu_sc as plsc

N = 10000
E = 320000
D = 128
DUMMY = N
N_PAD = 10240
NC, NS = 2, 16
NW = NC * NS
K = 128
NB = 80
HB = NB // 2
EPT = K * NB
E_PAD = EPT * NW
DEG_W = 128
RPT = N_PAD // NS

def _deg_body(rows_hbm, zeros_hbm, ones_hbm, degp_hbm, ridx, ones_v, accum):
    c = lax.axis_index("c")
    s = lax.axis_index("s")
    wid = c * NS + s
    pltpu.sync_copy(zeros_hbm.at[pl.ds(s * RPT, RPT)],
                    accum.at[pl.ds(s * RPT, RPT)])
    pltpu.sync_copy(ones_hbm, ones_v)
    pltpu.sync_copy(rows_hbm.at[pl.ds(wid * NB, NB)], ridx)
    plsc.subcore_barrier()

    def body(j, carry):
        pltpu.sync_copy(ones_v, accum.at[ridx.at[j]], add=True)
        return carry

    lax.fori_loop(0, NB, body, 0)
    plsc.subcore_barrier()
    pltpu.sync_copy(accum.at[pl.ds(s * RPT, RPT)],
                    degp_hbm.at[c, pl.ds(s * RPT, RPT)])


def _agg_body(xs_hbm, rows_hbm, cols_hbm, zeros_hbm, part_hbm,
              ridx, cidx, data0, data1, accum, sem0, sem1):
    c = lax.axis_index("c")
    s = lax.axis_index("s")
    wid = c * NS + s
    pltpu.sync_copy(zeros_hbm.at[pl.ds(s * RPT, RPT)],
                    accum.at[pl.ds(s * RPT, RPT)])
    plsc.subcore_barrier()

    for h in range(NB // HB):
        pltpu.sync_copy(rows_hbm.at[pl.ds(wid * NB + h * HB, HB)], ridx)
        pltpu.sync_copy(cols_hbm.at[pl.ds(wid * NB + h * HB, HB)], cidx)
        pltpu.async_copy(xs_hbm.at[ridx.at[0]], data0, sem0)

        def body(p, carry):
            j0 = 2 * p
            pltpu.async_copy(xs_hbm.at[ridx.at[j0 + 1]], data1, sem1)
            pltpu.make_async_copy(xs_hbm.at[ridx.at[j0]], data0, sem0).wait()
            pltpu.sync_copy(data0, accum.at[cidx.at[j0]], add=True)

            @pl.when(p + 1 < HB // 2)
            def _():
                pltpu.async_copy(xs_hbm.at[ridx.at[j0 + 2]], data0, sem0)

            pltpu.make_async_copy(xs_hbm.at[ridx.at[j0 + 1]], data1, sem1).wait()
            pltpu.sync_copy(data1, accum.at[cidx.at[j0 + 1]], add=True)
            return carry

        lax.fori_loop(0, HB // 2, body, 0)
    plsc.subcore_barrier()
    pltpu.sync_copy(accum.at[pl.ds(s * RPT, RPT)],
                    part_hbm.at[c, pl.ds(s * RPT, RPT)])


@functools.cache
def _sc_kernels():
    mesh = plsc.VectorSubcoreMesh(
        core_axis_name="c", subcore_axis_name="s",
        num_cores=NC, num_subcores=NS)
    deg_k = pl.kernel(
        _deg_body,
        out_type=jax.ShapeDtypeStruct((NC, N_PAD, DEG_W), jnp.float32),
        mesh=mesh,
        scratch_types=[
            pltpu.VMEM((NB, K), jnp.int32),
            pltpu.VMEM((K, DEG_W), jnp.float32),
            pltpu.VMEM_SHARED((N_PAD, DEG_W), jnp.float32),
        ],
    )
    agg_k = pl.kernel(
        _agg_body,
        out_type=jax.ShapeDtypeStruct((NC, N_PAD, D), jnp.float32),
        mesh=mesh,
        scratch_types=[
            pltpu.VMEM((HB, K), jnp.int32),
            pltpu.VMEM((HB, K), jnp.int32),
            pltpu.VMEM((K, D), jnp.float32),
            pltpu.VMEM((K, D), jnp.float32),
            pltpu.VMEM_SHARED((N_PAD, D), jnp.float32),
            pltpu.SemaphoreType.DMA,
            pltpu.SemaphoreType.DMA,
        ],
    )
    return deg_k, agg_k


def _mm_body(x_ref, w_ref, b_ref, xl_ref):
    xl_ref[...] = lax.dot_general(x_ref[...], w_ref[...],
                                  (((1,), (1,)), ((), ())),
                                  preferred_element_type=jnp.float32) + b_ref[...]


def _mm_call(x, W_lin, b_lin):
    return pl.pallas_call(
        _mm_body,
        grid=(N_PAD // 1024,),
        in_specs=[
            pl.BlockSpec((1024, D), lambda i: (i, 0)),
            pl.BlockSpec((D, D), lambda i: (0, 0)),
            pl.BlockSpec((1, D), lambda i: (0, 0)),
        ],
        out_specs=pl.BlockSpec((1024, D), lambda i: (i, 0)),
        out_shape=jax.ShapeDtypeStruct((N_PAD, D), jnp.float32),
    )(x, W_lin, b_lin[None, :])


def _scale_body(xl_ref, degp_ref, xs_ref):
    deg = degp_ref[0][:, 0:1] + degp_ref[1][:, 0:1] + 1.0
    dis = lax.rsqrt(deg)
    rid = pl.program_id(0) * 1024 + lax.broadcasted_iota(jnp.int32, (1024, 1), 0)
    xs_ref[...] = jnp.where(rid < N, dis * xl_ref[...], 0.0)


def _scale_call(xl, degp):
    return pl.pallas_call(
        _scale_body,
        grid=(N_PAD // 1024,),
        in_specs=[
            pl.BlockSpec((1024, D), lambda i: (i, 0)),
            pl.BlockSpec((NC, 1024, DEG_W), lambda i: (0, i, 0)),
        ],
        out_specs=pl.BlockSpec((1024, D), lambda i: (i, 0)),
        out_shape=jax.ShapeDtypeStruct((N_PAD, D), jnp.float32),
    )(xl, degp)


def _mlp_body(p_ref, xs_ref, degp_ref, w1_ref, b1_ref, w2_ref, b2_ref, y_ref):
    deg = degp_ref[0][:, 0:1] + degp_ref[1][:, 0:1] + 1.0
    dis = lax.rsqrt(deg)
    m = dis * (p_ref[0] + p_ref[1] + xs_ref[...])
    z = jnp.maximum(m, 0.0)
    h = lax.dot_general(z, w1_ref[...], (((1,), (1,)), ((), ())),
                        preferred_element_type=jnp.float32) + b1_ref[...]
    h = jnp.maximum(h, 0.0)
    y_ref[...] = lax.dot_general(h, w2_ref[...], (((1,), (1,)), ((), ())),
                                 preferred_element_type=jnp.float32) + b2_ref[...]


def _mlp_call(partials, xs, degp, W1, b1, W2, b2):
    return pl.pallas_call(
        _mlp_body,
        grid=(N // 1000,),
        in_specs=[
            pl.BlockSpec((NC, 1000, D), lambda i: (0, i, 0)),
            pl.BlockSpec((1000, D), lambda i: (i, 0)),
            pl.BlockSpec((NC, 1000, DEG_W), lambda i: (0, i, 0)),
            pl.BlockSpec((D, D), lambda i: (0, 0)),
            pl.BlockSpec((1, D), lambda i: (0, 0)),
            pl.BlockSpec((D, D), lambda i: (0, 0)),
            pl.BlockSpec((1, D), lambda i: (0, 0)),
        ],
        out_specs=pl.BlockSpec((1000, D), lambda i: (i, 0)),
        out_shape=jax.ShapeDtypeStruct((N, D), jnp.float32),
    )(partials, xs, degp, W1, b1[None, :], W2, b2[None, :])


def kernel(x, edge_index, W_lin, b_lin, W1, b1, W2, b2):
    pad = DUMMY + jnp.arange(E_PAD - E, dtype=jnp.int32) % (N_PAD - N)
    rows = jnp.concatenate([edge_index[0], pad]).reshape(E_PAD // K, K)
    cols = jnp.concatenate([edge_index[1], pad]).reshape(E_PAD // K, K)
    zeros = jnp.zeros((N_PAD, D), jnp.float32)
    ones = jnp.ones((K, DEG_W), jnp.float32)

    deg_kernel, agg_kernel = _sc_kernels()
    degp = deg_kernel(rows, zeros, ones)
    xl = _mm_call(x, W_lin, b_lin)
    xs = _scale_call(xl, degp)
    partials = agg_kernel(xs, rows, cols, zeros)
    return _mlp_call(partials, xs, degp, W1, b1, W2, b2)


# speedup vs baseline: 1.0070x; 1.0070x over previous
import functools

import jax
import jax.numpy as jnp
from jax import lax
from jax.experimental import pallas as pl
from jax.experimental.pallas import tpu as pltpu
from jax.experimental.pallas import tpu_sc as plsc

N = 10000
E = 320000
D = 128
DUMMY = N
N_PAD = 10240
NC, NS = 2, 16
NW = NC * NS
K = 128
NB = 80
HB = NB // 2
EPT = K * NB
E_PAD = EPT * NW
DEG_W = 128
RPT = N_PAD // NS

def _deg_body(rows_hbm, zeros_hbm, ones_hbm, degp_hbm, ridx, ones_v, accum):
    c = lax.axis_index("c")
    s = lax.axis_index("s")
    wid = c * NS + s
    pltpu.sync_copy(zeros_hbm.at[pl.ds(s * RPT, RPT)],
                    accum.at[pl.ds(s * RPT, RPT)])
    pltpu.sync_copy(ones_hbm, ones_v)
    pltpu.sync_copy(rows_hbm.at[pl.ds(wid * NB, NB)], ridx)
    plsc.subcore_barrier()

    def body(j, carry):
        pltpu.sync_copy(ones_v, accum.at[ridx.at[j]], add=True)
        return carry

    lax.fori_loop(0, NB, body, 0)
    plsc.subcore_barrier()
    pltpu.sync_copy(accum.at[pl.ds(s * RPT, RPT)],
                    degp_hbm.at[c, pl.ds(s * RPT, RPT)])


def _agg_body(xs_hbm, rows_hbm, cols_hbm, zeros_hbm, part_hbm,
              ridx, cidx, data0, data1, accum, sem0, sem1):
    c = lax.axis_index("c")
    s = lax.axis_index("s")
    wid = c * NS + s
    pltpu.sync_copy(zeros_hbm.at[pl.ds(s * RPT, RPT)],
                    accum.at[pl.ds(s * RPT, RPT)])
    plsc.subcore_barrier()

    for h in range(NB // HB):
        pltpu.sync_copy(rows_hbm.at[pl.ds(wid * NB + h * HB, HB)], ridx)
        pltpu.sync_copy(cols_hbm.at[pl.ds(wid * NB + h * HB, HB)], cidx)
        pltpu.async_copy(xs_hbm.at[ridx.at[0]], data0, sem0)

        def body(p, carry):
            j0 = 2 * p
            pltpu.async_copy(xs_hbm.at[ridx.at[j0 + 1]], data1, sem1)
            pltpu.make_async_copy(xs_hbm.at[ridx.at[j0]], data0, sem0).wait()
            pltpu.sync_copy(data0, accum.at[cidx.at[j0]], add=True)

            @pl.when(p + 1 < HB // 2)
            def _():
                pltpu.async_copy(xs_hbm.at[ridx.at[j0 + 2]], data0, sem0)

            pltpu.make_async_copy(xs_hbm.at[ridx.at[j0 + 1]], data1, sem1).wait()
            pltpu.sync_copy(data1, accum.at[cidx.at[j0 + 1]], add=True)
            return carry

        lax.fori_loop(0, HB // 2, body, 0)
    plsc.subcore_barrier()
    pltpu.sync_copy(accum.at[pl.ds(s * RPT, RPT)],
                    part_hbm.at[c, pl.ds(s * RPT, RPT)])


@functools.cache
def _sc_kernels():
    mesh = plsc.VectorSubcoreMesh(
        core_axis_name="c", subcore_axis_name="s",
        num_cores=NC, num_subcores=NS)
    deg_k = pl.kernel(
        _deg_body,
        out_type=jax.ShapeDtypeStruct((NC, N_PAD, DEG_W), jnp.float32),
        mesh=mesh,
        scratch_types=[
            pltpu.VMEM((NB, K), jnp.int32),
            pltpu.VMEM((K, DEG_W), jnp.float32),
            pltpu.VMEM_SHARED((N_PAD, DEG_W), jnp.float32),
        ],
    )
    agg_k = pl.kernel(
        _agg_body,
        out_type=jax.ShapeDtypeStruct((NC, N_PAD, D), jnp.float32),
        mesh=mesh,
        scratch_types=[
            pltpu.VMEM((HB, K), jnp.int32),
            pltpu.VMEM((HB, K), jnp.int32),
            pltpu.VMEM((K, D), jnp.float32),
            pltpu.VMEM((K, D), jnp.float32),
            pltpu.VMEM_SHARED((N_PAD, D), jnp.float32),
            pltpu.SemaphoreType.DMA,
            pltpu.SemaphoreType.DMA,
        ],
    )
    return deg_k, agg_k


def _lin_body(x_ref, w_ref, b_ref, degp_ref, xs_ref):
    deg = degp_ref[0][:, 0:1] + degp_ref[1][:, 0:1] + 1.0
    dis = lax.rsqrt(deg)
    xl = lax.dot_general(x_ref[...], w_ref[...], (((1,), (1,)), ((), ())),
                         preferred_element_type=jnp.float32) + b_ref[...]
    rid = pl.program_id(0) * 1024 + lax.broadcasted_iota(jnp.int32, (1024, 1), 0)
    xs_ref[...] = jnp.where(rid < N, dis * xl, 0.0)


def _lin_call(x, W_lin, b_lin, degp):
    return pl.pallas_call(
        _lin_body,
        grid=(N_PAD // 1024,),
        in_specs=[
            pl.BlockSpec((1024, D), lambda i: (i, 0)),
            pl.BlockSpec((D, D), lambda i: (0, 0)),
            pl.BlockSpec((1, D), lambda i: (0, 0)),
            pl.BlockSpec((NC, 1024, DEG_W), lambda i: (0, i, 0)),
        ],
        out_specs=pl.BlockSpec((1024, D), lambda i: (i, 0)),
        out_shape=jax.ShapeDtypeStruct((N_PAD, D), jnp.float32),
    )(x, W_lin, b_lin[None, :], degp)


def _mlp_body(p_ref, xs_ref, degp_ref, w1_ref, b1_ref, w2_ref, b2_ref, y_ref):
    deg = degp_ref[0][:, 0:1] + degp_ref[1][:, 0:1] + 1.0
    dis = lax.rsqrt(deg)
    m = dis * (p_ref[0] + p_ref[1] + xs_ref[...])
    z = jnp.maximum(m, 0.0)
    h = lax.dot_general(z, w1_ref[...], (((1,), (1,)), ((), ())),
                        preferred_element_type=jnp.float32) + b1_ref[...]
    h = jnp.maximum(h, 0.0)
    y_ref[...] = lax.dot_general(h, w2_ref[...], (((1,), (1,)), ((), ())),
                                 preferred_element_type=jnp.float32) + b2_ref[...]


def _mlp_call(partials, xs, degp, W1, b1, W2, b2):
    return pl.pallas_call(
        _mlp_body,
        grid=(N // 1000,),
        in_specs=[
            pl.BlockSpec((NC, 1000, D), lambda i: (0, i, 0)),
            pl.BlockSpec((1000, D), lambda i: (i, 0)),
            pl.BlockSpec((NC, 1000, DEG_W), lambda i: (0, i, 0)),
            pl.BlockSpec((D, D), lambda i: (0, 0)),
            pl.BlockSpec((1, D), lambda i: (0, 0)),
            pl.BlockSpec((D, D), lambda i: (0, 0)),
            pl.BlockSpec((1, D), lambda i: (0, 0)),
        ],
        out_specs=pl.BlockSpec((1000, D), lambda i: (i, 0)),
        out_shape=jax.ShapeDtypeStruct((N, D), jnp.float32),
    )(partials, xs, degp, W1, b1[None, :], W2, b2[None, :])


def kernel(x, edge_index, W_lin, b_lin, W1, b1, W2, b2):
    pad = DUMMY + jnp.arange(E_PAD - E, dtype=jnp.int32) % (N_PAD - N)
    rows = jnp.concatenate([edge_index[0], pad]).reshape(E_PAD // K, K)
    cols = jnp.concatenate([edge_index[1], pad]).reshape(E_PAD // K, K)
    zeros = jnp.zeros((N_PAD, D), jnp.float32)
    ones = jnp.ones((K, DEG_W), jnp.float32)

    deg_kernel, agg_kernel = _sc_kernels()
    degp = deg_kernel(rows, zeros, ones)
    xs = _lin_call(x, W_lin, b_lin, degp)
    partials = agg_kernel(xs, rows, cols, zeros)
    return _mlp_call(partials, xs, degp, W1, b1, W2, b2)
